# baseline (speedup 1.0000x reference)
import functools

import jax
import jax.numpy as jnp
from jax import lax
from jax.experimental import pallas as pl
from jax.experimental.pallas import tpu as pltpu
from jax.experimental.pallas import tpu_sc as plsc

N_NODES = 10000
N_EDGES = 320000
D = 128
DE = 16

_NPART = 5
_EP = N_EDGES // _NPART

_NC = 2
_NS = 16
_NW = _NC * _NS
_EPW = _EP // _NW
_CH = 80
_NCHUNK = _EPW // _CH
_NB = 5

_BE = 2560
_NBLK = _EP // _BE


def _node_proj_body(x_ref, wa_ref, wb_ref, xa_ref, xb_ref):
    x = x_ref[...]
    xa_ref[...] = jnp.dot(x, wa_ref[...], preferred_element_type=jnp.float32)
    xb_ref[...] = jnp.dot(x, wb_ref[...], preferred_element_type=jnp.float32)


def _node_proj(x, w1a, w1b):
    return pl.pallas_call(
        _node_proj_body,
        out_shape=(
            jax.ShapeDtypeStruct((N_NODES, D), jnp.float32),
            jax.ShapeDtypeStruct((N_NODES, D), jnp.float32),
        ),
    )(x, w1a, w1b)


def _edge_gather(xa, xb, src, dst):
    mesh = plsc.VectorSubcoreMesh(core_axis_name="c", subcore_axis_name="s")

    @functools.partial(
        pl.kernel,
        mesh=mesh,
        out_type=jax.ShapeDtypeStruct((_EP, D), jnp.float32),
        scratch_types=(
            [pltpu.VMEM((_CH,), jnp.int32)] * (2 * _NB)
            + [pltpu.VMEM((_CH, D), jnp.float32)] * _NB
            + [pltpu.SemaphoreType.DMA] * (4 * _NB)
        ),
    )
    def gather_k(xa_hbm, xb_hbm, src_hbm, dst_hbm, g_hbm, *bufs):
        wid = lax.axis_index("s") * _NC + lax.axis_index("c")
        SI = bufs[0:_NB]
        DI = bufs[_NB:2 * _NB]
        RA = bufs[2 * _NB:3 * _NB]
        sems = bufs[3 * _NB:]
        IXS = sems[0:_NB]
        GSA = sems[_NB:2 * _NB]
        GSB = sems[2 * _NB:3 * _NB]
        STS = sems[3 * _NB:4 * _NB]

        def idx_copies(g, b):
            base = wid * _EPW + g * _CH
            return (
                pltpu.make_async_copy(src_hbm.at[pl.ds(base, _CH)], SI[b], IXS[b]),
                pltpu.make_async_copy(dst_hbm.at[pl.ds(base, _CH)], DI[b], IXS[b]),
            )

        def gath_a(b):
            return pltpu.make_async_copy(xa_hbm.at[SI[b]], RA[b], GSA[b])

        def gath_b_drain(b):
            return pltpu.make_async_copy(xb_hbm.at[DI[b]], RA[b], GSB[b])

        def store_copy(g, b):
            base = wid * _EPW + g * _CH
            return pltpu.make_async_copy(RA[b], g_hbm.at[pl.ds(base, _CH)], STS[b])

        def fire(cs):
            for c in cs:
                c.start()

        def drain(cs):
            for c in cs:
                c.wait()

        fire(idx_copies(0, 0))
        drain(idx_copies(0, 0))
        gath_a(0).start()
        fire(idx_copies(1, 1))
        fire(idx_copies(2, 2))

        def step(g, b):
            b1, bm1, bm2, b3 = (b + 1) % _NB, (b - 1) % _NB, (b - 2) % _NB, (b + 3) % _NB

            @pl.when(g + 1 <= _NCHUNK - 1)
            def _(g=g, b1=b1):
                drain(idx_copies(g + 1, b1))
                gath_a(b1).start()

            gath_a(b).wait()
            pltpu.async_copy(xb_hbm.at[DI[b]], RA[b], GSB[b], add=True)

            @pl.when(g >= 1)
            def _(g=g, bm1=bm1):
                gath_b_drain(bm1).wait()
                store_copy(g - 1, bm1).start()

            @pl.when(g >= 2)
            def _(g=g, bm2=bm2):
                store_copy(g - 2, bm2).wait()

            @pl.when(g + 3 <= _NCHUNK - 1)
            def _(g=g, b3=b3):
                fire(idx_copies(g + 3, b3))

        def body(c, carry):
            for b in range(_NB):
                step(_NB * c + b, b)
            return carry

        lax.fori_loop(0, _NCHUNK // _NB, body, 0)

        last = _NCHUNK - 1
        lb = last % _NB
        gath_b_drain(lb).wait()
        store_copy(last, lb).start()
        store_copy(last - 1, (last - 1) % _NB).wait()
        store_copy(last, lb).wait()

    return gather_k(xa, xb, src, dst)


def _edge_partial_body(g_ref, ea_ref, w1c_ref, b1_ref, w2_ref,
                       b2_ref, out_ref, acc_ref):
    step = pl.program_id(0)

    @pl.when(step == 0)
    def _():
        acc_ref[...] = jnp.zeros_like(acc_ref)

    m1 = g_ref[...] + b1_ref[...]
    m1 += jnp.dot(ea_ref[...], w1c_ref[...], preferred_element_type=jnp.float32)
    m1 = jnp.maximum(m1, 0.0).astype(jnp.bfloat16)
    m = jnp.dot(m1, w2_ref[...], preferred_element_type=jnp.float32)
    m = jnp.maximum(m + b2_ref[...], 0.0)
    acc_ref[...] += jnp.sum(m, axis=0, keepdims=True)

    @pl.when(step == _NBLK - 1)
    def _():
        out_ref[...] = acc_ref[...]


def _edge_partial(g, ea, w1c, b1, w2, b2):
    fixed = lambda i: (0, 0)
    return pl.pallas_call(
        _edge_partial_body,
        grid=(_NBLK,),
        in_specs=[
            pl.BlockSpec((_BE, D), lambda i: (i, 0)),
            pl.BlockSpec((_BE, DE), lambda i: (i, 0)),
            pl.BlockSpec((DE, D), fixed),
            pl.BlockSpec((1, D), fixed),
            pl.BlockSpec((D, D), fixed),
            pl.BlockSpec((1, D), fixed),
        ],
        out_specs=pl.BlockSpec((1, D), fixed),
        out_shape=jax.ShapeDtypeStruct((1, D), jnp.float32),
        scratch_shapes=[pltpu.VMEM((1, D), jnp.float32)],
    )(g, ea, w1c, b1, w2, b2)


def _final_body(sp_ref, wn_ref, bn_ref, wg_ref, bg_ref, u_ref, out_ref):
    s = jnp.sum(sp_ref[...], axis=0, keepdims=True)
    snf = jnp.dot(s, wn_ref[...], preferred_element_type=jnp.float32)
    snf += jnp.float32(N_NODES) * bn_ref[...]
    g = jnp.dot(u_ref[...], wg_ref[:D, :], preferred_element_type=jnp.float32)
    g += jnp.dot(snf, wg_ref[D:, :], preferred_element_type=jnp.float32)
    out_ref[...] = jnp.maximum(g + bg_ref[...], 0.0)


def _final(sp, wn, bn, wg, bg, u):
    return pl.pallas_call(
        _final_body,
        out_shape=jax.ShapeDtypeStruct((1, D), jnp.float32),
    )(sp, wn, bn, wg, bg, u)


def kernel(x, edge_index, edge_attr, u, W1, b1, W2, b2, Wn, bn, Wg, bg):
    src = edge_index[0]
    dst = edge_index[1]
    w1a, w1b, w1c = W1[:D], W1[D:2 * D], W1[2 * D:]
    xa, xb = _node_proj(x, w1a, w1b)
    b1r = b1.reshape(1, D)
    b2r = b2.reshape(1, D)
    w2b = W2.astype(jnp.bfloat16)
    parts = []
    for p in range(_NPART):
        lo, hi = p * _EP, (p + 1) * _EP
        g = _edge_gather(xa, xb, src[lo:hi], dst[lo:hi])
        parts.append(_edge_partial(g, edge_attr[lo:hi], w1c, b1r,
                                   w2b, b2r))
    sp = jnp.concatenate(parts + [jnp.zeros((8 - _NPART, D), jnp.float32)],
                         axis=0)
    return _final(sp, Wn, bn.reshape(1, D), Wg, bg.reshape(1, D), u)


# speedup vs baseline: 1.0186x; 1.0186x over previous
import functools

import jax
import jax.numpy as jnp
from jax import lax
from jax.experimental import pallas as pl
from jax.experimental.pallas import tpu as pltpu
from jax.experimental.pallas import tpu_sc as plsc

N_NODES = 10000
N_EDGES = 320000
D = 128
DE = 16

_NPART = 5
_EP = N_EDGES // _NPART

_NC = 2
_NS = 16
_NW = _NC * _NS
_EPW = _EP // _NW
_CH = 40
_NCHUNK = _EPW // _CH
_NB = 5

_BE = 2560
_NBLK = _EP // _BE


def _node_proj_body(x_ref, wa_ref, wb_ref, xa_ref, xb_ref):
    x = x_ref[...]
    xa_ref[...] = jnp.dot(x, wa_ref[...], preferred_element_type=jnp.float32)
    xb_ref[...] = jnp.dot(x, wb_ref[...], preferred_element_type=jnp.float32)


def _node_proj(x, w1a, w1b):
    return pl.pallas_call(
        _node_proj_body,
        out_shape=(
            jax.ShapeDtypeStruct((N_NODES, D), jnp.float32),
            jax.ShapeDtypeStruct((N_NODES, D), jnp.float32),
        ),
    )(x, w1a, w1b)


def _edge_gather(xa, xb, src, dst):
    mesh = plsc.VectorSubcoreMesh(core_axis_name="c", subcore_axis_name="s")

    @functools.partial(
        pl.kernel,
        mesh=mesh,
        out_type=jax.ShapeDtypeStruct((_EP, D), jnp.float32),
        scratch_types=(
            [pltpu.VMEM((_CH,), jnp.int32)] * (2 * _NB)
            + [pltpu.VMEM((_CH, D), jnp.float32)] * _NB
            + [pltpu.SemaphoreType.DMA] * (4 * _NB)
        ),
    )
    def gather_k(xa_hbm, xb_hbm, src_hbm, dst_hbm, g_hbm, *bufs):
        wid = lax.axis_index("s") * _NC + lax.axis_index("c")
        SI = bufs[0:_NB]
        DI = bufs[_NB:2 * _NB]
        RA = bufs[2 * _NB:3 * _NB]
        sems = bufs[3 * _NB:]
        IXS = sems[0:_NB]
        GSA = sems[_NB:2 * _NB]
        GSB = sems[2 * _NB:3 * _NB]
        STS = sems[3 * _NB:4 * _NB]

        def idx_copies(g, b):
            base = wid * _EPW + g * _CH
            return (
                pltpu.make_async_copy(src_hbm.at[pl.ds(base, _CH)], SI[b], IXS[b]),
                pltpu.make_async_copy(dst_hbm.at[pl.ds(base, _CH)], DI[b], IXS[b]),
            )

        def gath_a(b):
            return pltpu.make_async_copy(xa_hbm.at[SI[b]], RA[b], GSA[b])

        def gath_b_drain(b):
            return pltpu.make_async_copy(xb_hbm.at[DI[b]], RA[b], GSB[b])

        def store_copy(g, b):
            base = wid * _EPW + g * _CH
            return pltpu.make_async_copy(RA[b], g_hbm.at[pl.ds(base, _CH)], STS[b])

        def fire(cs):
            for c in cs:
                c.start()

        def drain(cs):
            for c in cs:
                c.wait()

        fire(idx_copies(0, 0))
        drain(idx_copies(0, 0))
        gath_a(0).start()
        fire(idx_copies(1, 1))
        fire(idx_copies(2, 2))

        def step(g, b):
            b1, bm1, bm2, b3 = (b + 1) % _NB, (b - 1) % _NB, (b - 2) % _NB, (b + 3) % _NB

            @pl.when(g + 1 <= _NCHUNK - 1)
            def _(g=g, b1=b1):
                drain(idx_copies(g + 1, b1))
                gath_a(b1).start()

            gath_a(b).wait()
            pltpu.async_copy(xb_hbm.at[DI[b]], RA[b], GSB[b], add=True)

            @pl.when(g >= 1)
            def _(g=g, bm1=bm1):
                gath_b_drain(bm1).wait()
                store_copy(g - 1, bm1).start()

            @pl.when(g >= 2)
            def _(g=g, bm2=bm2):
                store_copy(g - 2, bm2).wait()

            @pl.when(g + 3 <= _NCHUNK - 1)
            def _(g=g, b3=b3):
                fire(idx_copies(g + 3, b3))

        def body(c, carry):
            for b in range(_NB):
                step(_NB * c + b, b)
            return carry

        lax.fori_loop(0, _NCHUNK // _NB, body, 0)

        last = _NCHUNK - 1
        lb = last % _NB
        gath_b_drain(lb).wait()
        store_copy(last, lb).start()
        store_copy(last - 1, (last - 1) % _NB).wait()
        store_copy(last, lb).wait()

    return gather_k(xa, xb, src, dst)


def _edge_partial_body(g_ref, ea_ref, w1c_ref, b1_ref, w2_ref,
                       b2_ref, out_ref, acc_ref):
    step = pl.program_id(0)

    @pl.when(step == 0)
    def _():
        acc_ref[...] = jnp.zeros_like(acc_ref)

    m1 = g_ref[...] + b1_ref[...]
    m1 += jnp.dot(ea_ref[...], w1c_ref[...], preferred_element_type=jnp.float32)
    m1 = jnp.maximum(m1, 0.0).astype(jnp.bfloat16)
    m = jnp.dot(m1, w2_ref[...], preferred_element_type=jnp.float32)
    m = jnp.maximum(m + b2_ref[...], 0.0)
    acc_ref[...] += jnp.sum(m, axis=0, keepdims=True)

    @pl.when(step == _NBLK - 1)
    def _():
        out_ref[...] = acc_ref[...]


def _edge_partial(g, ea, w1c, b1, w2, b2):
    fixed = lambda i: (0, 0)
    return pl.pallas_call(
        _edge_partial_body,
        grid=(_NBLK,),
        in_specs=[
            pl.BlockSpec((_BE, D), lambda i: (i, 0)),
            pl.BlockSpec((_BE, DE), lambda i: (i, 0)),
            pl.BlockSpec((DE, D), fixed),
            pl.BlockSpec((1, D), fixed),
            pl.BlockSpec((D, D), fixed),
            pl.BlockSpec((1, D), fixed),
        ],
        out_specs=pl.BlockSpec((1, D), fixed),
        out_shape=jax.ShapeDtypeStruct((1, D), jnp.float32),
        scratch_shapes=[pltpu.VMEM((1, D), jnp.float32)],
    )(g, ea, w1c, b1, w2, b2)


def _final_body(sp_ref, wn_ref, bn_ref, wg_ref, bg_ref, u_ref, out_ref):
    s = jnp.sum(sp_ref[...], axis=0, keepdims=True)
    snf = jnp.dot(s, wn_ref[...], preferred_element_type=jnp.float32)
    snf += jnp.float32(N_NODES) * bn_ref[...]
    g = jnp.dot(u_ref[...], wg_ref[:D, :], preferred_element_type=jnp.float32)
    g += jnp.dot(snf, wg_ref[D:, :], preferred_element_type=jnp.float32)
    out_ref[...] = jnp.maximum(g + bg_ref[...], 0.0)


def _final(sp, wn, bn, wg, bg, u):
    return pl.pallas_call(
        _final_body,
        out_shape=jax.ShapeDtypeStruct((1, D), jnp.float32),
    )(sp, wn, bn, wg, bg, u)


def kernel(x, edge_index, edge_attr, u, W1, b1, W2, b2, Wn, bn, Wg, bg):
    src = edge_index[0]
    dst = edge_index[1]
    w1a, w1b, w1c = W1[:D], W1[D:2 * D], W1[2 * D:]
    xa, xb = _node_proj(x, w1a, w1b)
    b1r = b1.reshape(1, D)
    b2r = b2.reshape(1, D)
    w2b = W2.astype(jnp.bfloat16)
    parts = []
    for p in range(_NPART):
        lo, hi = p * _EP, (p + 1) * _EP
        g = _edge_gather(xa, xb, src[lo:hi], dst[lo:hi])
        parts.append(_edge_partial(g, edge_attr[lo:hi], w1c, b1r,
                                   w2b, b2r))
    sp = jnp.concatenate(parts + [jnp.zeros((8 - _NPART, D), jnp.float32)],
                         axis=0)
    return _final(sp, Wn, bn.reshape(1, D), Wg, bg.reshape(1, D), u)
